# baseline (speedup 1.0000x reference)
import functools

import jax
import jax.numpy as jnp
from jax import lax
from jax.experimental import pallas as pl
from jax.experimental.pallas import tpu as pltpu
from jax.experimental.pallas import tpu_sc as plsc

_B = 4 * 4096
_D = 1024
_NC = 2
_NS = 16
_NW = _NC * _NS
_B_PER_W = _B // _NW
_CHUNK = 16
_NCHUNKS = _B_PER_W // _CHUNK
_NBUF = 4


def _gather_body(table_hbm, x_hbm, out_hbm, idx_v, bufs, gsems, osems):
    wid = lax.axis_index("s") * _NC + lax.axis_index("c")
    base = wid * _B_PER_W
    pltpu.sync_copy(
        x_hbm.at[wid // 8, pl.ds((wid % 8) * _B_PER_W, _B_PER_W)], idx_v)

    def gather(g):
        b = g % _NBUF
        return pltpu.async_copy(
            table_hbm.at[idx_v.at[pl.ds(g * _CHUNK, _CHUNK)]],
            bufs[b], gsems[b])

    gathers = [None] * _NCHUNKS
    outs = [None] * _NCHUNKS
    for g in range(min(_NBUF - 1, _NCHUNKS)):
        gathers[g] = gather(g)
    for g in range(_NCHUNKS):
        b = g % _NBUF
        gathers[g].wait()
        outs[g] = pltpu.async_copy(
            bufs[b], out_hbm.at[pl.ds(base + g * _CHUNK, _CHUNK)], osems[b])
        nxt = g + _NBUF - 1
        if nxt < _NCHUNKS:
            prev = nxt - _NBUF
            if prev >= 0:
                outs[prev].wait()
            gathers[nxt] = gather(nxt)
    for g in range(max(0, _NCHUNKS - _NBUF), _NCHUNKS):
        outs[g].wait()


_sc_gather = functools.partial(
    pl.kernel,
    out_type=jax.ShapeDtypeStruct((_B, _D), jnp.float32),
    mesh=plsc.VectorSubcoreMesh(core_axis_name="c", subcore_axis_name="s"),
    scratch_types=[
        pltpu.VMEM((_B_PER_W,), jnp.int32),
        [pltpu.VMEM((_CHUNK, _D), jnp.float32) for _ in range(_NBUF)],
        [pltpu.SemaphoreType.DMA for _ in range(_NBUF)],
        [pltpu.SemaphoreType.DMA for _ in range(_NBUF)],
    ],
)(_gather_body)


def kernel(x, table):
    out = _sc_gather(table, x.astype(jnp.int32))
    return out.reshape(x.shape + (_D,))
